# Initial kernel scaffold; baseline (speedup 1.0000x reference)
import jax
import jax.numpy as jnp
from jax.experimental import pallas as pl


def kernel(x, edge_index, edge_attr, W1, b1, W2, b2, w_attn, W_msg, b_msg, W_self, b_self, conv_w, conv_b):
    raise NotImplementedError("write your pallas kernel here")


# speedup vs baseline: 17.2607x; 17.2607x over previous
import functools

import jax
import jax.numpy as jnp
from jax import lax
from jax.experimental import pallas as pl
from jax.experimental.pallas import tpu as pltpu
from jax.experimental.pallas import tpu_sc as plsc

NC, NS = 2, 16
NW = NC * NS
EB = 1000
NB = 1000
CH = 40
AW = 144


def _edge_tc(ea, W1, b1, W2, b2, wa, Wme, bm, ex_o, cc_o):
    enc = jnp.maximum(ea[...] @ W1[...] + b1[...], 0.0)
    enc = jnp.maximum(enc @ W2[...] + b2[...], 0.0)
    logit = jnp.clip(enc @ wa[...], -60.0, 60.0)
    ex = jnp.exp(logit)
    ex_o[...] = ex
    cc_o[...] = ex * (enc @ Wme[...] + bm[...])


def _node_tc(h, Wc, bs, xwc_o, sp_o):
    a = h[...]
    a0 = a[0] @ Wc[...]
    a1 = a[1] @ Wc[...]
    xwc_o[...] = jnp.concatenate([a0[:, :64], a1[:, :64]], axis=1)
    sp_o[...] = jnp.stack([a0[:, 64:] + bs[...], a1[:, 64:] + bs[...]])


def _final_tc(part, sp, CCm, cb, y_o):
    p = part[...]
    agg = p[0] + p[1]
    denom = jnp.clip(agg[:, 128:129], 1e-12, None)
    s = sp[...]
    s0 = jnp.maximum(agg[:, 0:64] / denom + s[0], 0.0)
    s1 = jnp.maximum(agg[:, 64:128] / denom + s[1], 0.0)
    S = jnp.concatenate([s0, s1], axis=1)
    Y = S @ CCm[...] + cb[...]
    y_o[...] = jnp.stack([Y[:, :64], Y[:, 64:]])


def _sc_body(xwc, src, dst, exv, cc, part,
             agg_sh, src_v, dst_v, ex_v, cc_v, g_v, s_v, sem,
             *, n_nodes, e_total):
    c = lax.axis_index("c")
    s = lax.axis_index("s")
    tile = c * NS + s
    per_tile = e_total // NW
    rows_pt = n_nodes // NS
    nchunks = per_tile // CH

    zero16 = jnp.zeros((16,), jnp.float32)

    def zrow(e, carry):
        for q in range(AW // 16):
            s_v[e, pl.ds(q * 16, 16)] = zero16
        return carry

    lax.fori_loop(0, CH, zrow, 0)

    zc = 25

    def zcopy(j, carry):
        pltpu.sync_copy(s_v.at[pl.ds(0, zc)],
                        agg_sh.at[pl.ds(s * rows_pt + j * zc, zc)])
        return carry

    lax.fori_loop(0, rows_pt // zc, zcopy, 0)
    plsc.subcore_barrier()

    first = lax.iota(jnp.int32, 16) == 0
    ebase = tile * per_tile

    def chunk(k, carry):
        kb = ebase + k * CH
        pltpu.sync_copy(src.at[pl.ds(kb, CH)], src_v)
        pltpu.sync_copy(dst.at[pl.ds(kb, CH)], dst_v)
        pltpu.sync_copy(exv.at[pl.ds(kb, CH)], ex_v)
        pltpu.sync_copy(cc.at[pl.ds(kb, CH)], cc_v)
        pltpu.async_copy(xwc.at[src_v], g_v, sem).wait()

        def ebody(e, ecarry):
            exs = plsc.load_gather(ex_v, [jnp.full((16,), e, jnp.int32)])
            s_v[e, pl.ds(128, 16)] = jnp.where(first, exs, 0.0)
            for q in range(4):
                ccq = cc_v[e, pl.ds(q * 16, 16)]
                for t in range(2):
                    col = t * 64 + q * 16
                    m = jnp.maximum(g_v[e, pl.ds(col, 16)] * exs + ccq, 0.0)
                    s_v[e, pl.ds(col, 16)] = m
            return ecarry

        lax.fori_loop(0, CH, ebody, 0)
        pltpu.sync_copy(s_v, agg_sh.at[dst_v], add=True)
        return carry

    lax.fori_loop(0, nchunks, chunk, 0)

    plsc.subcore_barrier()
    pltpu.sync_copy(agg_sh.at[pl.ds(s * rows_pt, rows_pt)],
                    part.at[c, pl.ds(s * rows_pt, rows_pt)])


def kernel(x, edge_index, edge_attr, W1, b1, W2, b2, w_attn, W_msg, b_msg,
           W_self, b_self, conv_w, conv_b):
    B_, T_, N_, F_ = x.shape
    E_ = edge_attr.shape[0]
    D_ = edge_attr.shape[1]
    H_ = W_self.shape[1]
    src = edge_index[0].astype(jnp.int32)
    dst = edge_index[1].astype(jnp.int32)
    h2 = x.reshape(B_ * T_, N_, F_)
    Wm_x = W_msg[:F_]
    Wm_e = W_msg[F_:]

    exv, cc = pl.pallas_call(
        _edge_tc,
        grid=(E_ // EB,),
        in_specs=[
            pl.BlockSpec((EB, D_), lambda i: (i, 0)),
            pl.BlockSpec((D_, H_), lambda i: (0, 0)),
            pl.BlockSpec((1, H_), lambda i: (0, 0)),
            pl.BlockSpec((H_, H_), lambda i: (0, 0)),
            pl.BlockSpec((1, H_), lambda i: (0, 0)),
            pl.BlockSpec((H_, 1), lambda i: (0, 0)),
            pl.BlockSpec((H_, H_), lambda i: (0, 0)),
            pl.BlockSpec((1, H_), lambda i: (0, 0)),
        ],
        out_specs=[pl.BlockSpec((EB, 1), lambda i: (i, 0)),
                   pl.BlockSpec((EB, H_), lambda i: (i, 0))],
        out_shape=[jax.ShapeDtypeStruct((E_, 1), jnp.float32),
                   jax.ShapeDtypeStruct((E_, H_), jnp.float32)],
    )(edge_attr, W1, b1.reshape(1, -1), W2, b2.reshape(1, -1), w_attn,
      Wm_e, b_msg.reshape(1, -1))

    Wc = jnp.concatenate([Wm_x, W_self], axis=1)
    xwc, sp = pl.pallas_call(
        _node_tc,
        grid=(N_ // NB,),
        in_specs=[pl.BlockSpec((2, NB, F_), lambda i: (0, i, 0)),
                  pl.BlockSpec((F_, 2 * H_), lambda i: (0, 0)),
                  pl.BlockSpec((1, H_), lambda i: (0, 0))],
        out_specs=[pl.BlockSpec((NB, 2 * H_), lambda i: (i, 0)),
                   pl.BlockSpec((2, NB, H_), lambda i: (0, i, 0))],
        out_shape=[jax.ShapeDtypeStruct((N_, 2 * H_), jnp.float32),
                   jax.ShapeDtypeStruct((2, N_, H_), jnp.float32)],
    )(h2, Wc, b_self.reshape(1, -1))

    mesh = plsc.VectorSubcoreMesh(core_axis_name="c", subcore_axis_name="s",
                                  num_cores=NC, num_subcores=NS)
    sc_fn = pl.kernel(
        functools.partial(_sc_body, n_nodes=N_, e_total=E_),
        out_type=jax.ShapeDtypeStruct((NC, N_, AW), jnp.float32),
        mesh=mesh,
        scratch_types=[
            pltpu.VMEM_SHARED((N_, AW), jnp.float32),
            pltpu.VMEM((CH,), jnp.int32),
            pltpu.VMEM((CH,), jnp.int32),
            pltpu.VMEM((CH,), jnp.float32),
            pltpu.VMEM((CH, 64), jnp.float32),
            pltpu.VMEM((CH, 128), jnp.float32),
            pltpu.VMEM((CH, AW), jnp.float32),
            pltpu.SemaphoreType.DMA,
        ],
        compiler_params=pltpu.CompilerParams(use_tc_tiling_on_sc=False,
                                             needs_layout_passes=False),
    )
    part = sc_fn(xwc, src, dst, exv.reshape(E_), cc)

    C0 = conv_w[:, :, 0].T
    C1 = conv_w[:, :, 1].T
    C2 = conv_w[:, :, 2].T
    CCm = jnp.concatenate([jnp.concatenate([C1, C0], axis=1),
                           jnp.concatenate([C2, C1], axis=1)], axis=0)
    cb = jnp.concatenate([conv_b, conv_b]).reshape(1, -1)
    y2 = pl.pallas_call(
        _final_tc,
        grid=(N_ // NB,),
        in_specs=[pl.BlockSpec((2, NB, AW), lambda i: (0, i, 0)),
                  pl.BlockSpec((2, NB, H_), lambda i: (0, i, 0)),
                  pl.BlockSpec((2 * H_, 2 * H_), lambda i: (0, 0)),
                  pl.BlockSpec((1, 2 * H_), lambda i: (0, 0))],
        out_specs=pl.BlockSpec((2, NB, H_), lambda i: (0, i, 0)),
        out_shape=jax.ShapeDtypeStruct((2, N_, H_), jnp.float32),
    )(part, sp, CCm, cb)
    return y2.reshape(B_, T_, N_, H_)
